# baseline (speedup 1.0000x reference)
import functools
import jax
import jax.numpy as jnp
from jax import lax
from jax.experimental import pallas as pl
from jax.experimental.pallas import tpu as pltpu
from jax.experimental.pallas import tpu_sc as plsc

NUM_MOLS = 100
DEPTH = 4
N = 10000
E = 320000
H = 128
MAXNB = 32

NC, NS = 2, 16
NW = NC * NS
CA = 4
CE = 128
A_CHUNKS = N // CA
E_CHUNKS = E // CE
A_PER_W = -(-A_CHUNKS // NW)
E_PER_W = -(-E_CHUNKS // NW)


def _sc_gather_sum_body(msg_hbm, a2b_hbm, out_hbm, idx_v, rows_v, acc_v, sem):
    wid = lax.axis_index("c") * NS + lax.axis_index("s")
    lo = wid * A_PER_W
    hi = jnp.minimum(lo + A_PER_W, A_CHUNKS)

    def chunk(c, carry):
        pltpu.sync_copy(a2b_hbm.at[pl.ds(c * (CA * MAXNB), CA * MAXNB)], idx_v)
        pltpu.async_copy(msg_hbm.at[idx_v], rows_v, sem).wait()
        for j in range(CA):
            for h in range(H // 16):
                def kbody(k, acc):
                    return acc + rows_v[j * MAXNB + k, pl.ds(h * 16, 16)]
                acc = lax.fori_loop(0, MAXNB, kbody,
                                    jnp.zeros((16,), jnp.float32), unroll=8)
                acc_v[j, pl.ds(h * 16, 16)] = acc
        pltpu.sync_copy(acc_v, out_hbm.at[pl.ds(c * CA, CA)])
        return carry

    lax.fori_loop(lo, hi, chunk, 0)


def _make_sc_gather_sum():
    mesh = plsc.VectorSubcoreMesh(core_axis_name="c", subcore_axis_name="s")
    return pl.kernel(
        _sc_gather_sum_body,
        out_type=jax.ShapeDtypeStruct((N, H), jnp.float32),
        mesh=mesh,
        scratch_types=[
            pltpu.VMEM((CA * MAXNB,), jnp.int32),
            pltpu.VMEM((CA * MAXNB, H), jnp.float32),
            pltpu.VMEM((CA, H), jnp.float32),
            pltpu.SemaphoreType.DMA,
        ],
    )


def _sc_edge_body(amsg_hbm, msg_hbm, b2a_hbm, b2revb_hbm, out_hbm,
                  ia_v, ir_v, am_v, rev_v, sem_a, sem_b):
    wid = lax.axis_index("c") * NS + lax.axis_index("s")
    lo = wid * E_PER_W
    hi = jnp.minimum(lo + E_PER_W, E_CHUNKS)

    def chunk(c, carry):
        pltpu.sync_copy(b2a_hbm.at[pl.ds(c * CE, CE)], ia_v)
        pltpu.sync_copy(b2revb_hbm.at[pl.ds(c * CE, CE)], ir_v)
        cp_a = pltpu.async_copy(amsg_hbm.at[ia_v], am_v, sem_a)
        cp_b = pltpu.async_copy(msg_hbm.at[ir_v], rev_v, sem_b)
        cp_a.wait()
        cp_b.wait()

        def jbody(j, carry2):
            for h in range(H // 16):
                sl = pl.ds(h * 16, 16)
                am_v[j, sl] = am_v[j, sl] - rev_v[j, sl]
            return carry2
        lax.fori_loop(0, CE, jbody, 0, unroll=4)
        pltpu.sync_copy(am_v, out_hbm.at[pl.ds(c * CE, CE)])
        return carry

    lax.fori_loop(lo, hi, chunk, 0)


def _make_sc_edge():
    mesh = plsc.VectorSubcoreMesh(core_axis_name="c", subcore_axis_name="s")
    return pl.kernel(
        _sc_edge_body,
        out_type=jax.ShapeDtypeStruct((E, H), jnp.float32),
        mesh=mesh,
        scratch_types=[
            pltpu.VMEM((CE,), jnp.int32),
            pltpu.VMEM((CE,), jnp.int32),
            pltpu.VMEM((CE, H), jnp.float32),
            pltpu.VMEM((CE, H), jnp.float32),
            pltpu.SemaphoreType.DMA,
            pltpu.SemaphoreType.DMA,
        ],
    )


_BP = 3200


def _prologue_body(fb_ref, wia_ref, wib_ref, wibb_ref, bibb_ref, wjbb_ref,
                   bjbb_ref, inp_ref, msg_ref, ebond_ref):
    fb = fb_ref[...]
    inp = jnp.dot(fb, wia_ref[...], preferred_element_type=jnp.float32)
    inp_ref[...] = inp
    msg_ref[...] = jnp.maximum(inp, 0.0)
    fbb = jnp.dot(fb, wib_ref[...], preferred_element_type=jnp.float32)
    gate = jax.nn.sigmoid(
        jnp.dot(fbb, wjbb_ref[...], preferred_element_type=jnp.float32)
        + bjbb_ref[...])
    val = (jnp.dot(fbb, wibb_ref[...], preferred_element_type=jnp.float32)
           + bibb_ref[...])
    ebond_ref[0] = jnp.sum(gate * val, axis=0, keepdims=True)


def _prologue(f_bonds, W_i_atom, W_i_bond, W_ib, b_ib, W_jb, b_jb):
    grid = (E // _BP,)
    BF = f_bonds.shape[1]
    full = lambda shape: pl.BlockSpec(shape, lambda i: (0,) * len(shape))
    out = pl.pallas_call(
        _prologue_body,
        grid=grid,
        in_specs=[
            pl.BlockSpec((_BP, BF), lambda i: (i, 0)),
            full((BF, H)), full((BF, H)),
            full((H, H)), full((1, H)), full((H, H)), full((1, H)),
        ],
        out_specs=[
            pl.BlockSpec((_BP, H), lambda i: (i, 0)),
            pl.BlockSpec((_BP, H), lambda i: (i, 0)),
            pl.BlockSpec((1, 1, H), lambda i: (i, 0, 0)),
        ],
        out_shape=[
            jax.ShapeDtypeStruct((E, H), jnp.float32),
            jax.ShapeDtypeStruct((E, H), jnp.float32),
            jax.ShapeDtypeStruct((NUM_MOLS, 1, H), jnp.float32),
        ],
    )(f_bonds, W_i_atom, W_i_bond, W_ib, b_ib.reshape(1, H),
      W_jb, b_jb.reshape(1, H))
    inp, msg0, ebond = out
    return inp, msg0, ebond.reshape(NUM_MOLS, H)


_BM = 3200


def _update_body(tmp_ref, inp_ref, wh_ref, out_ref):
    out_ref[...] = jnp.maximum(
        inp_ref[...]
        + jnp.dot(tmp_ref[...], wh_ref[...], preferred_element_type=jnp.float32),
        0.0)


def _update(tmp, inp, W_h):
    return pl.pallas_call(
        _update_body,
        grid=(E // _BM,),
        in_specs=[
            pl.BlockSpec((_BM, H), lambda i: (i, 0)),
            pl.BlockSpec((_BM, H), lambda i: (i, 0)),
            pl.BlockSpec((H, H), lambda i: (0, 0)),
        ],
        out_specs=pl.BlockSpec((_BM, H), lambda i: (i, 0)),
        out_shape=jax.ShapeDtypeStruct((E, H), jnp.float32),
    )(tmp, inp, W_h)


def _epilogue_body(fa_ref, am_ref, wo_ref, bo_ref, wiaa_ref, biaa_ref,
                   wjaa_ref, bjaa_ref, sel_ref, eatom_ref):
    ah = jnp.maximum(
        jnp.dot(fa_ref[...], wo_ref[:H], preferred_element_type=jnp.float32)
        + jnp.dot(am_ref[...], wo_ref[H:], preferred_element_type=jnp.float32)
        + bo_ref[...],
        0.0)
    gate = jax.nn.sigmoid(
        jnp.dot(ah, wjaa_ref[...], preferred_element_type=jnp.float32)
        + bjaa_ref[...])
    val = (jnp.dot(ah, wiaa_ref[...], preferred_element_type=jnp.float32)
           + biaa_ref[...])
    eatom_ref[...] = jnp.dot(sel_ref[...], gate * val,
                             preferred_element_type=jnp.float32)


def _epilogue(f_atoms, a_msg, W_o, b_o, W_ia, b_ia, W_ja, b_ja):
    AF = f_atoms.shape[1]
    sel = jnp.repeat(jnp.eye(NUM_MOLS, dtype=jnp.float32), N // NUM_MOLS,
                     axis=1)
    full = lambda shape: pl.BlockSpec(shape, lambda: (0,) * len(shape))
    return pl.pallas_call(
        _epilogue_body,
        in_specs=[
            full((N, AF)), full((N, H)), full((AF + H, H)), full((1, H)),
            full((H, H)), full((1, H)), full((H, H)), full((1, H)),
            full((NUM_MOLS, N)),
        ],
        out_specs=full((NUM_MOLS, H)),
        out_shape=jax.ShapeDtypeStruct((NUM_MOLS, H), jnp.float32),
    )(f_atoms, a_msg, W_o, b_o.reshape(1, H), W_ia, b_ia.reshape(1, H),
      W_ja, b_ja.reshape(1, H), sel)


@jax.jit
def _run(f_atoms, f_bonds, a2b, b2a, b2revb,
         W_i_atom, W_h, W_o, b_o, W_ia, b_ia, W_ja, b_ja,
         W_i_bond, W_ib, b_ib, W_jb, b_jb):
    a2b_flat = a2b.astype(jnp.int32).reshape(-1)
    b2a = b2a.astype(jnp.int32)
    b2revb = b2revb.astype(jnp.int32)

    gather_sum = _make_sc_gather_sum()
    edge_combine = _make_sc_edge()

    inp, message, e_bond = _prologue(f_bonds, W_i_atom, W_i_bond,
                                     W_ib, b_ib, W_jb, b_jb)
    for _ in range(DEPTH - 1):
        a_msg = gather_sum(message, a2b_flat)
        tmp = edge_combine(a_msg, message, b2a, b2revb)
        message = _update(tmp, inp, W_h)
    a_msg = gather_sum(message, a2b_flat)
    e_atom = _epilogue(f_atoms, a_msg, W_o, b_o, W_ia, b_ia, W_ja, b_ja)
    return jnp.concatenate([e_atom, e_bond], axis=1)


def kernel(f_atoms, f_bonds, a2b, b2a, b2revb,
           W_i_atom, W_h, W_o, b_o, W_ia, b_ia, W_ja, b_ja,
           W_i_bond, W_ib, b_ib, W_jb, b_jb):
    return _run(f_atoms, f_bonds, a2b, b2a, b2revb,
                W_i_atom, W_h, W_o, b_o, W_ia, b_ia, W_ja, b_ja,
                W_i_bond, W_ib, b_ib, W_jb, b_jb)


# speedup vs baseline: 1.5398x; 1.5398x over previous
import functools
import jax
import jax.numpy as jnp
from jax import lax
from jax.experimental import pallas as pl
from jax.experimental.pallas import tpu as pltpu
from jax.experimental.pallas import tpu_sc as plsc

NUM_MOLS = 100
DEPTH = 4
N = 10000
E = 320000
H = 128
MAXNB = 32

NC, NS = 2, 16
NW = NC * NS
CA = 4
CE = 128
A_CHUNKS = N // CA
E_CHUNKS = E // CE
A_PER_W = -(-A_CHUNKS // NW)
E_PER_W = -(-E_CHUNKS // NW)


NBUF = 2
CI = CA * MAXNB


def _sc_gather_sum_body(msg_hbm, a2b_hbm, out_hbm, idx_v, rows_v, acc_v,
                        g0, g1, w0, w1):
    gsem = (g0, g1)
    wsem = (w0, w1)
    wid = lax.axis_index("c") * NS + lax.axis_index("s")
    lo = wid * A_PER_W
    hi = jnp.minimum(lo + A_PER_W, A_CHUNKS)

    pltpu.sync_copy(a2b_hbm.at[pl.ds(lo * CI, A_PER_W * CI)], idx_v)

    def gather(ordinal, b):
        pltpu.async_copy(msg_hbm.at[idx_v.at[pl.ds(ordinal * CI, CI)]],
                         rows_v.at[b], gsem[b])

    for b in range(NBUF):
        @pl.when(lo + b < hi)
        def _():
            gather(b, b)

    def outer(i, carry):
        for b in range(NBUF):
            ordinal = i * NBUF + b
            c = lo + ordinal

            @pl.when(c < hi)
            def _():
                pltpu.make_async_copy(
                    msg_hbm.at[pl.ds(0, CI)], rows_v.at[b], gsem[b]).wait()

                @pl.when(ordinal >= NBUF)
                def _():
                    pltpu.make_async_copy(
                        acc_v.at[b], out_hbm.at[pl.ds(0, CA)], wsem[b]).wait()

                for j in range(CA):
                    for h in range(H // 16):
                        def kbody(k, acc):
                            return acc + rows_v[b, j * MAXNB + k,
                                                pl.ds(h * 16, 16)]
                        acc = lax.fori_loop(0, MAXNB, kbody,
                                            jnp.zeros((16,), jnp.float32),
                                            unroll=8)
                        acc_v[b, j, pl.ds(h * 16, 16)] = acc
                pltpu.async_copy(acc_v.at[b], out_hbm.at[pl.ds(c * CA, CA)],
                                 wsem[b])

                @pl.when(c + NBUF < hi)
                def _():
                    gather(ordinal + NBUF, b)
        return carry

    lax.fori_loop(0, -(-A_PER_W // NBUF), outer, 0)
    for b in range(NBUF):
        pltpu.make_async_copy(acc_v.at[b], out_hbm.at[pl.ds(0, CA)],
                              wsem[b]).wait()


def _make_sc_gather_sum():
    mesh = plsc.VectorSubcoreMesh(core_axis_name="c", subcore_axis_name="s")
    return pl.kernel(
        _sc_gather_sum_body,
        out_type=jax.ShapeDtypeStruct((N, H), jnp.float32),
        mesh=mesh,
        scratch_types=[
            pltpu.VMEM((A_PER_W * CI,), jnp.int32),
            pltpu.VMEM((NBUF, CI, H), jnp.float32),
            pltpu.VMEM((NBUF, CA, H), jnp.float32),
            pltpu.SemaphoreType.DMA,
            pltpu.SemaphoreType.DMA,
            pltpu.SemaphoreType.DMA,
            pltpu.SemaphoreType.DMA,
        ],
    )


def _sc_edge_body(amsg_hbm, msg_hbm, b2a_hbm, b2revb_hbm, out_hbm,
                  ia_v, ir_v, am_v, rev_v, out_v,
                  ga0, ga1, gr0, gr1, w0, w1):
    gasem = (ga0, ga1)
    grsem = (gr0, gr1)
    wsem = (w0, w1)
    wid = lax.axis_index("c") * NS + lax.axis_index("s")
    lo = wid * E_PER_W
    hi = jnp.minimum(lo + E_PER_W, E_CHUNKS)

    pltpu.sync_copy(b2a_hbm.at[pl.ds(lo * CE, E_PER_W * CE)], ia_v)
    pltpu.sync_copy(b2revb_hbm.at[pl.ds(lo * CE, E_PER_W * CE)], ir_v)

    def gathers(ordinal, b):
        sl = pl.ds(ordinal * CE, CE)
        pltpu.async_copy(amsg_hbm.at[ia_v.at[sl]], am_v.at[b], gasem[b])
        pltpu.async_copy(msg_hbm.at[ir_v.at[sl]], rev_v.at[b], grsem[b])

    for b in range(NBUF):
        @pl.when(lo + b < hi)
        def _():
            gathers(b, b)

    def outer(i, carry):
        for b in range(NBUF):
            ordinal = i * NBUF + b
            c = lo + ordinal

            @pl.when(c < hi)
            def _():
                pltpu.make_async_copy(
                    amsg_hbm.at[pl.ds(0, CE)], am_v.at[b], gasem[b]).wait()
                pltpu.make_async_copy(
                    msg_hbm.at[pl.ds(0, CE)], rev_v.at[b], grsem[b]).wait()

                @pl.when(ordinal >= NBUF)
                def _():
                    pltpu.make_async_copy(
                        out_v.at[b], out_hbm.at[pl.ds(0, CE)], wsem[b]).wait()

                def jbody(j, carry2):
                    for h in range(H // 16):
                        sl = pl.ds(h * 16, 16)
                        out_v[b, j, sl] = am_v[b, j, sl] - rev_v[b, j, sl]
                    return carry2
                lax.fori_loop(0, CE, jbody, 0, unroll=8)

                pltpu.async_copy(out_v.at[b], out_hbm.at[pl.ds(c * CE, CE)],
                                 wsem[b])

                @pl.when(c + NBUF < hi)
                def _():
                    gathers(ordinal + NBUF, b)
        return carry

    lax.fori_loop(0, -(-E_PER_W // NBUF), outer, 0)
    for b in range(NBUF):
        pltpu.make_async_copy(out_v.at[b], out_hbm.at[pl.ds(0, CE)],
                              wsem[b]).wait()


def _make_sc_edge():
    mesh = plsc.VectorSubcoreMesh(core_axis_name="c", subcore_axis_name="s")
    return pl.kernel(
        _sc_edge_body,
        out_type=jax.ShapeDtypeStruct((E, H), jnp.float32),
        mesh=mesh,
        scratch_types=[
            pltpu.VMEM((E_PER_W * CE,), jnp.int32),
            pltpu.VMEM((E_PER_W * CE,), jnp.int32),
            pltpu.VMEM((NBUF, CE, H), jnp.float32),
            pltpu.VMEM((NBUF, CE, H), jnp.float32),
            pltpu.VMEM((NBUF, CE, H), jnp.float32),
            pltpu.SemaphoreType.DMA,
            pltpu.SemaphoreType.DMA,
            pltpu.SemaphoreType.DMA,
            pltpu.SemaphoreType.DMA,
            pltpu.SemaphoreType.DMA,
            pltpu.SemaphoreType.DMA,
        ],
    )


_BP = 3200


def _prologue_body(fb_ref, wia_ref, wib_ref, wibb_ref, bibb_ref, wjbb_ref,
                   bjbb_ref, inp_ref, msg_ref, ebond_ref):
    fb = fb_ref[...]
    inp = jnp.dot(fb, wia_ref[...], preferred_element_type=jnp.float32)
    inp_ref[...] = inp
    msg_ref[...] = jnp.maximum(inp, 0.0)
    fbb = jnp.dot(fb, wib_ref[...], preferred_element_type=jnp.float32)
    gate = jax.nn.sigmoid(
        jnp.dot(fbb, wjbb_ref[...], preferred_element_type=jnp.float32)
        + bjbb_ref[...])
    val = (jnp.dot(fbb, wibb_ref[...], preferred_element_type=jnp.float32)
           + bibb_ref[...])
    ebond_ref[0] = jnp.sum(gate * val, axis=0, keepdims=True)


def _prologue(f_bonds, W_i_atom, W_i_bond, W_ib, b_ib, W_jb, b_jb):
    grid = (E // _BP,)
    BF = f_bonds.shape[1]
    full = lambda shape: pl.BlockSpec(shape, lambda i: (0,) * len(shape))
    out = pl.pallas_call(
        _prologue_body,
        grid=grid,
        in_specs=[
            pl.BlockSpec((_BP, BF), lambda i: (i, 0)),
            full((BF, H)), full((BF, H)),
            full((H, H)), full((1, H)), full((H, H)), full((1, H)),
        ],
        out_specs=[
            pl.BlockSpec((_BP, H), lambda i: (i, 0)),
            pl.BlockSpec((_BP, H), lambda i: (i, 0)),
            pl.BlockSpec((1, 1, H), lambda i: (i, 0, 0)),
        ],
        out_shape=[
            jax.ShapeDtypeStruct((E, H), jnp.float32),
            jax.ShapeDtypeStruct((E, H), jnp.float32),
            jax.ShapeDtypeStruct((NUM_MOLS, 1, H), jnp.float32),
        ],
    )(f_bonds, W_i_atom, W_i_bond, W_ib, b_ib.reshape(1, H),
      W_jb, b_jb.reshape(1, H))
    inp, msg0, ebond = out
    return inp, msg0, ebond.reshape(NUM_MOLS, H)


_BM = 3200


def _update_body(tmp_ref, inp_ref, wh_ref, out_ref):
    out_ref[...] = jnp.maximum(
        inp_ref[...]
        + jnp.dot(tmp_ref[...], wh_ref[...], preferred_element_type=jnp.float32),
        0.0)


def _update(tmp, inp, W_h):
    return pl.pallas_call(
        _update_body,
        grid=(E // _BM,),
        in_specs=[
            pl.BlockSpec((_BM, H), lambda i: (i, 0)),
            pl.BlockSpec((_BM, H), lambda i: (i, 0)),
            pl.BlockSpec((H, H), lambda i: (0, 0)),
        ],
        out_specs=pl.BlockSpec((_BM, H), lambda i: (i, 0)),
        out_shape=jax.ShapeDtypeStruct((E, H), jnp.float32),
    )(tmp, inp, W_h)


def _epilogue_body(fa_ref, am_ref, wo_ref, bo_ref, wiaa_ref, biaa_ref,
                   wjaa_ref, bjaa_ref, sel_ref, eatom_ref):
    ah = jnp.maximum(
        jnp.dot(fa_ref[...], wo_ref[:H], preferred_element_type=jnp.float32)
        + jnp.dot(am_ref[...], wo_ref[H:], preferred_element_type=jnp.float32)
        + bo_ref[...],
        0.0)
    gate = jax.nn.sigmoid(
        jnp.dot(ah, wjaa_ref[...], preferred_element_type=jnp.float32)
        + bjaa_ref[...])
    val = (jnp.dot(ah, wiaa_ref[...], preferred_element_type=jnp.float32)
           + biaa_ref[...])
    eatom_ref[...] = jnp.dot(sel_ref[...], gate * val,
                             preferred_element_type=jnp.float32)


def _epilogue(f_atoms, a_msg, W_o, b_o, W_ia, b_ia, W_ja, b_ja):
    AF = f_atoms.shape[1]
    sel = jnp.repeat(jnp.eye(NUM_MOLS, dtype=jnp.float32), N // NUM_MOLS,
                     axis=1)
    full = lambda shape: pl.BlockSpec(shape, lambda: (0,) * len(shape))
    return pl.pallas_call(
        _epilogue_body,
        in_specs=[
            full((N, AF)), full((N, H)), full((AF + H, H)), full((1, H)),
            full((H, H)), full((1, H)), full((H, H)), full((1, H)),
            full((NUM_MOLS, N)),
        ],
        out_specs=full((NUM_MOLS, H)),
        out_shape=jax.ShapeDtypeStruct((NUM_MOLS, H), jnp.float32),
    )(f_atoms, a_msg, W_o, b_o.reshape(1, H), W_ia, b_ia.reshape(1, H),
      W_ja, b_ja.reshape(1, H), sel)


@jax.jit
def _run(f_atoms, f_bonds, a2b, b2a, b2revb,
         W_i_atom, W_h, W_o, b_o, W_ia, b_ia, W_ja, b_ja,
         W_i_bond, W_ib, b_ib, W_jb, b_jb):
    pad_a = NW * A_PER_W * CI - N * MAXNB
    pad_e = NW * E_PER_W * CE - E
    a2b_flat = jnp.concatenate(
        [a2b.astype(jnp.int32).reshape(-1), jnp.zeros((pad_a,), jnp.int32)])
    b2a = jnp.concatenate(
        [b2a.astype(jnp.int32), jnp.zeros((pad_e,), jnp.int32)])
    b2revb = jnp.concatenate(
        [b2revb.astype(jnp.int32), jnp.zeros((pad_e,), jnp.int32)])

    gather_sum = _make_sc_gather_sum()
    edge_combine = _make_sc_edge()

    inp, message, e_bond = _prologue(f_bonds, W_i_atom, W_i_bond,
                                     W_ib, b_ib, W_jb, b_jb)
    for _ in range(DEPTH - 1):
        a_msg = gather_sum(message, a2b_flat)
        tmp = edge_combine(a_msg, message, b2a, b2revb)
        message = _update(tmp, inp, W_h)
    a_msg = gather_sum(message, a2b_flat)
    e_atom = _epilogue(f_atoms, a_msg, W_o, b_o, W_ia, b_ia, W_ja, b_ja)
    return jnp.concatenate([e_atom, e_bond], axis=1)


def kernel(f_atoms, f_bonds, a2b, b2a, b2revb,
           W_i_atom, W_h, W_o, b_o, W_ia, b_ia, W_ja, b_ja,
           W_i_bond, W_ib, b_ib, W_jb, b_jb):
    return _run(f_atoms, f_bonds, a2b, b2a, b2revb,
                W_i_atom, W_h, W_o, b_o, W_ia, b_ia, W_ja, b_ja,
                W_i_bond, W_ib, b_ib, W_jb, b_jb)


# speedup vs baseline: 1.8898x; 1.2273x over previous
import functools
import jax
import jax.numpy as jnp
from jax import lax
from jax.experimental import pallas as pl
from jax.experimental.pallas import tpu as pltpu
from jax.experimental.pallas import tpu_sc as plsc

NUM_MOLS = 100
DEPTH = 4
N = 10000
E = 320000
H = 128
MAXNB = 32

NC, NS = 2, 16
NW = NC * NS
CA = 4
CE = 128
A_CHUNKS = N // CA
E_CHUNKS = E // CE
A_PER_W = -(-A_CHUNKS // NW)
E_PER_W = -(-E_CHUNKS // NW)


NBUF = 2
CI = CA * MAXNB


def _sc_gather_sum_body(msg_hbm, a2b_hbm, out_hbm, idx_v, rows_v, acc_v,
                        g0, g1, w0, w1):
    gsem = (g0, g1)
    wsem = (w0, w1)
    wid = lax.axis_index("c") * NS + lax.axis_index("s")
    lo = wid * A_PER_W
    hi = jnp.minimum(lo + A_PER_W, A_CHUNKS)

    pltpu.sync_copy(a2b_hbm.at[pl.ds(lo * CI, A_PER_W * CI)], idx_v)

    def gather(ordinal, b):
        pltpu.async_copy(msg_hbm.at[idx_v.at[pl.ds(ordinal * CI, CI)]],
                         rows_v.at[b], gsem[b])

    for b in range(NBUF):
        @pl.when(lo + b < hi)
        def _():
            gather(b, b)

    def outer(i, carry):
        for b in range(NBUF):
            ordinal = i * NBUF + b
            c = lo + ordinal

            @pl.when(c < hi)
            def _():
                pltpu.make_async_copy(
                    msg_hbm.at[pl.ds(0, CI)], rows_v.at[b], gsem[b]).wait()

                @pl.when(ordinal >= NBUF)
                def _():
                    pltpu.make_async_copy(
                        acc_v.at[b], out_hbm.at[pl.ds(0, CA)], wsem[b]).wait()

                for j in range(CA):
                    for h in range(H // 16):
                        def kbody(k, acc):
                            return acc + rows_v[b, j * MAXNB + k,
                                                pl.ds(h * 16, 16)]
                        acc = lax.fori_loop(0, MAXNB, kbody,
                                            jnp.zeros((16,), jnp.float32),
                                            unroll=8)
                        acc_v[b, j, pl.ds(h * 16, 16)] = acc
                pltpu.async_copy(acc_v.at[b], out_hbm.at[pl.ds(c * CA, CA)],
                                 wsem[b])

                @pl.when(c + NBUF < hi)
                def _():
                    gather(ordinal + NBUF, b)
        return carry

    lax.fori_loop(0, -(-A_PER_W // NBUF), outer, 0)
    for b in range(NBUF):
        pltpu.make_async_copy(acc_v.at[b], out_hbm.at[pl.ds(0, CA)],
                              wsem[b]).wait()


def _make_sc_gather_sum():
    mesh = plsc.VectorSubcoreMesh(core_axis_name="c", subcore_axis_name="s")
    return pl.kernel(
        _sc_gather_sum_body,
        out_type=jax.ShapeDtypeStruct((N, H), jnp.float32),
        mesh=mesh,
        scratch_types=[
            pltpu.VMEM((A_PER_W * CI,), jnp.int32),
            pltpu.VMEM((NBUF, CI, H), jnp.float32),
            pltpu.VMEM((NBUF, CA, H), jnp.float32),
            pltpu.SemaphoreType.DMA,
            pltpu.SemaphoreType.DMA,
            pltpu.SemaphoreType.DMA,
            pltpu.SemaphoreType.DMA,
        ],
    )


CE3 = 80
E3_CHUNKS = E // CE3
E3_PER_W = E3_CHUNKS // NW


def _sc_edge_body(amw_hbm, msgw_hbm, inp_hbm, b2a_hbm, b2revb_hbm, out_hbm,
                  ia_v, ir_v, am_v, rev_v, inp_v, out_v,
                  ga0, ga1, gr0, gr1, gi0, gi1, w0, w1):
    gasem = (ga0, ga1)
    grsem = (gr0, gr1)
    gisem = (gi0, gi1)
    wsem = (w0, w1)
    wid = lax.axis_index("c") * NS + lax.axis_index("s")
    lo = wid * E3_PER_W

    pltpu.sync_copy(b2a_hbm.at[pl.ds(lo * CE3, E3_PER_W * CE3)], ia_v)
    pltpu.sync_copy(b2revb_hbm.at[pl.ds(lo * CE3, E3_PER_W * CE3)], ir_v)

    def fetch(ordinal, b):
        sl = pl.ds(ordinal * CE3, CE3)
        c = lo + ordinal
        pltpu.async_copy(amw_hbm.at[ia_v.at[sl]], am_v.at[b], gasem[b])
        pltpu.async_copy(msgw_hbm.at[ir_v.at[sl]], rev_v.at[b], grsem[b])
        pltpu.async_copy(inp_hbm.at[pl.ds(c * CE3, CE3)], inp_v.at[b],
                         gisem[b])

    for b in range(NBUF):
        fetch(b, b)

    def outer(i, carry):
        for b in range(NBUF):
            ordinal = i * NBUF + b
            c = lo + ordinal

            pltpu.make_async_copy(
                amw_hbm.at[pl.ds(0, CE3)], am_v.at[b], gasem[b]).wait()
            pltpu.make_async_copy(
                msgw_hbm.at[pl.ds(0, CE3)], rev_v.at[b], grsem[b]).wait()
            pltpu.make_async_copy(
                inp_hbm.at[pl.ds(0, CE3)], inp_v.at[b], gisem[b]).wait()

            @pl.when(ordinal >= NBUF)
            def _():
                pltpu.make_async_copy(
                    out_v.at[b], out_hbm.at[pl.ds(0, CE3)], wsem[b]).wait()

            def jbody(j, carry2):
                for h in range(H // 16):
                    sl = pl.ds(h * 16, 16)
                    out_v[b, j, sl] = jnp.maximum(
                        inp_v[b, j, sl] + am_v[b, j, sl] - rev_v[b, j, sl],
                        0.0)
                return carry2
            lax.fori_loop(0, CE3, jbody, 0, unroll=8)

            pltpu.async_copy(out_v.at[b], out_hbm.at[pl.ds(c * CE3, CE3)],
                             wsem[b])

            @pl.when(ordinal + NBUF < E3_PER_W)
            def _():
                fetch(ordinal + NBUF, b)
        return carry

    lax.fori_loop(0, E3_PER_W // NBUF, outer, 0)
    ordinal = E3_PER_W - 1
    c = lo + ordinal
    b = ordinal % NBUF
    pltpu.make_async_copy(
        amw_hbm.at[pl.ds(0, CE3)], am_v.at[b], gasem[b]).wait()
    pltpu.make_async_copy(
        msgw_hbm.at[pl.ds(0, CE3)], rev_v.at[b], grsem[b]).wait()
    pltpu.make_async_copy(
        inp_hbm.at[pl.ds(0, CE3)], inp_v.at[b], gisem[b]).wait()
    pltpu.make_async_copy(
        out_v.at[b], out_hbm.at[pl.ds(0, CE3)], wsem[b]).wait()

    def jbody(j, carry2):
        for h in range(H // 16):
            sl = pl.ds(h * 16, 16)
            out_v[b, j, sl] = jnp.maximum(
                inp_v[b, j, sl] + am_v[b, j, sl] - rev_v[b, j, sl], 0.0)
        return carry2
    lax.fori_loop(0, CE3, jbody, 0, unroll=8)
    pltpu.async_copy(out_v.at[b], out_hbm.at[pl.ds(c * CE3, CE3)], wsem[b])

    for b in range(NBUF):
        pltpu.make_async_copy(out_v.at[b], out_hbm.at[pl.ds(0, CE3)],
                              wsem[b]).wait()


def _make_sc_edge():
    mesh = plsc.VectorSubcoreMesh(core_axis_name="c", subcore_axis_name="s")
    return pl.kernel(
        _sc_edge_body,
        out_type=jax.ShapeDtypeStruct((E, H), jnp.float32),
        mesh=mesh,
        scratch_types=[
            pltpu.VMEM((E3_PER_W * CE3,), jnp.int32),
            pltpu.VMEM((E3_PER_W * CE3,), jnp.int32),
            pltpu.VMEM((NBUF, CE3, H), jnp.float32),
            pltpu.VMEM((NBUF, CE3, H), jnp.float32),
            pltpu.VMEM((NBUF, CE3, H), jnp.float32),
            pltpu.VMEM((NBUF, CE3, H), jnp.float32),
            pltpu.SemaphoreType.DMA,
            pltpu.SemaphoreType.DMA,
            pltpu.SemaphoreType.DMA,
            pltpu.SemaphoreType.DMA,
            pltpu.SemaphoreType.DMA,
            pltpu.SemaphoreType.DMA,
            pltpu.SemaphoreType.DMA,
            pltpu.SemaphoreType.DMA,
        ],
    )


_BP = 3200


def _prologue_body(fb_ref, wia_ref, wib_ref, wibb_ref, bibb_ref, wjbb_ref,
                   bjbb_ref, inp_ref, msg_ref, ebond_ref):
    fb = fb_ref[...]
    inp = jnp.dot(fb, wia_ref[...], preferred_element_type=jnp.float32)
    inp_ref[...] = inp
    msg_ref[...] = jnp.maximum(inp, 0.0)
    fbb = jnp.dot(fb, wib_ref[...], preferred_element_type=jnp.float32)
    gate = jax.nn.sigmoid(
        jnp.dot(fbb, wjbb_ref[...], preferred_element_type=jnp.float32)
        + bjbb_ref[...])
    val = (jnp.dot(fbb, wibb_ref[...], preferred_element_type=jnp.float32)
           + bibb_ref[...])
    ebond_ref[0] = jnp.sum(gate * val, axis=0, keepdims=True)


def _prologue(f_bonds, W_i_atom, W_i_bond, W_ib, b_ib, W_jb, b_jb):
    grid = (E // _BP,)
    BF = f_bonds.shape[1]
    full = lambda shape: pl.BlockSpec(shape, lambda i: (0,) * len(shape))
    out = pl.pallas_call(
        _prologue_body,
        grid=grid,
        in_specs=[
            pl.BlockSpec((_BP, BF), lambda i: (i, 0)),
            full((BF, H)), full((BF, H)),
            full((H, H)), full((1, H)), full((H, H)), full((1, H)),
        ],
        out_specs=[
            pl.BlockSpec((_BP, H), lambda i: (i, 0)),
            pl.BlockSpec((_BP, H), lambda i: (i, 0)),
            pl.BlockSpec((1, 1, H), lambda i: (i, 0, 0)),
        ],
        out_shape=[
            jax.ShapeDtypeStruct((E, H), jnp.float32),
            jax.ShapeDtypeStruct((E, H), jnp.float32),
            jax.ShapeDtypeStruct((NUM_MOLS, 1, H), jnp.float32),
        ],
    )(f_bonds, W_i_atom, W_i_bond, W_ib, b_ib.reshape(1, H),
      W_jb, b_jb.reshape(1, H))
    inp, msg0, ebond = out
    return inp, msg0, ebond.reshape(NUM_MOLS, H)


def _mm_body(x_ref, w_ref, out_ref):
    out_ref[...] = jnp.dot(x_ref[...], w_ref[...],
                           preferred_element_type=jnp.float32)


def _mm(x, w, bm):
    R = x.shape[0]
    return pl.pallas_call(
        _mm_body,
        grid=(R // bm,),
        in_specs=[
            pl.BlockSpec((bm, H), lambda i: (i, 0)),
            pl.BlockSpec((H, H), lambda i: (0, 0)),
        ],
        out_specs=pl.BlockSpec((bm, H), lambda i: (i, 0)),
        out_shape=jax.ShapeDtypeStruct((R, H), jnp.float32),
    )(x, w)


def _epilogue_body(fa_ref, am_ref, wo_ref, bo_ref, wiaa_ref, biaa_ref,
                   wjaa_ref, bjaa_ref, sel_ref, eatom_ref):
    ah = jnp.maximum(
        jnp.dot(fa_ref[...], wo_ref[:H], preferred_element_type=jnp.float32)
        + jnp.dot(am_ref[...], wo_ref[H:], preferred_element_type=jnp.float32)
        + bo_ref[...],
        0.0)
    gate = jax.nn.sigmoid(
        jnp.dot(ah, wjaa_ref[...], preferred_element_type=jnp.float32)
        + bjaa_ref[...])
    val = (jnp.dot(ah, wiaa_ref[...], preferred_element_type=jnp.float32)
           + biaa_ref[...])
    eatom_ref[...] = jnp.dot(sel_ref[...], gate * val,
                             preferred_element_type=jnp.float32)


def _epilogue(f_atoms, a_msg, W_o, b_o, W_ia, b_ia, W_ja, b_ja):
    AF = f_atoms.shape[1]
    sel = jnp.repeat(jnp.eye(NUM_MOLS, dtype=jnp.float32), N // NUM_MOLS,
                     axis=1)
    full = lambda shape: pl.BlockSpec(shape, lambda: (0,) * len(shape))
    return pl.pallas_call(
        _epilogue_body,
        in_specs=[
            full((N, AF)), full((N, H)), full((AF + H, H)), full((1, H)),
            full((H, H)), full((1, H)), full((H, H)), full((1, H)),
            full((NUM_MOLS, N)),
        ],
        out_specs=full((NUM_MOLS, H)),
        out_shape=jax.ShapeDtypeStruct((NUM_MOLS, H), jnp.float32),
    )(f_atoms, a_msg, W_o, b_o.reshape(1, H), W_ia, b_ia.reshape(1, H),
      W_ja, b_ja.reshape(1, H), sel)


@jax.jit
def _run(f_atoms, f_bonds, a2b, b2a, b2revb,
         W_i_atom, W_h, W_o, b_o, W_ia, b_ia, W_ja, b_ja,
         W_i_bond, W_ib, b_ib, W_jb, b_jb):
    pad_a = NW * A_PER_W * CI - N * MAXNB
    a2b_flat = jnp.concatenate(
        [a2b.astype(jnp.int32).reshape(-1), jnp.zeros((pad_a,), jnp.int32)])
    b2a = b2a.astype(jnp.int32)
    b2revb = b2revb.astype(jnp.int32)

    gather_sum = _make_sc_gather_sum()
    edge_combine = _make_sc_edge()

    inp, message, e_bond = _prologue(f_bonds, W_i_atom, W_i_bond,
                                     W_ib, b_ib, W_jb, b_jb)
    for _ in range(DEPTH - 1):
        a_msg = gather_sum(message, a2b_flat)
        msgw = _mm(message, W_h, 3200)
        amw = _mm(a_msg, W_h, 2000)
        message = edge_combine(amw, msgw, inp, b2a, b2revb)
    a_msg = gather_sum(message, a2b_flat)
    e_atom = _epilogue(f_atoms, a_msg, W_o, b_o, W_ia, b_ia, W_ja, b_ja)
    return jnp.concatenate([e_atom, e_bond], axis=1)


def kernel(f_atoms, f_bonds, a2b, b2a, b2revb,
           W_i_atom, W_h, W_o, b_o, W_ia, b_ia, W_ja, b_ja,
           W_i_bond, W_ib, b_ib, W_jb, b_jb):
    return _run(f_atoms, f_bonds, a2b, b2a, b2revb,
                W_i_atom, W_h, W_o, b_o, W_ia, b_ia, W_ja, b_ja,
                W_i_bond, W_ib, b_ib, W_jb, b_jb)
